# baseline (speedup 1.0000x reference)
import functools

import jax
import jax.numpy as jnp
from jax import lax
from jax.experimental import pallas as pl
from jax.experimental.pallas import tpu as pltpu
from jax.experimental.pallas import tpu_sc as plsc

OH = 224
OW = 224
H = 224
W = 224
C = 192
B = 8
P = B * OH * OW
NW = 32
PT = P // NW
K = 64
NB = PT // K
L = 16
CAP = 4 * K
CH = 32

assert K % L == 0 and CAP % CH == 0 and PT % K == 0 and NB % 2 == 0


def _body(xd_hbm, yd_hbm, tab_hbm, out_hbm,
          xd_v, yd_v, id_v, wa_v, wb_v, wc_v, wd_v,
          pa_v, pb_v, pc_v, pd_v,
          cidx_v, crows_v, out_v, gsem0, gsem1, wsem):
    wid = lax.axis_index("s") * 2 + lax.axis_index("c")
    tile_base = wid * PT
    batch_off = (wid // 4) * (H * W)
    iota = lax.iota(jnp.int32, L)
    gsems = (gsem0, gsem1)

    for p in range(2):
        for q in range(CAP // L):
            cidx_v[p, pl.ds(q * L, L)] = jnp.zeros((L,), jnp.int32)

    def stage_compute_fire(blk, p):
        base = tile_base + blk * K
        pltpu.sync_copy(xd_hbm.at[pl.ds(base, K)], xd_v.at[p])
        pltpu.sync_copy(yd_hbm.at[pl.ds(base, K)], yd_v.at[p])
        running = jnp.int32(0)
        for g in range(K // L):
            s16 = pl.ds(g * L, L)
            x = 0.5 * (xd_v[p, s16] + 1.0) * 224.0
            y = 0.5 * (yd_v[p, s16] + 1.0) * 224.0
            x0 = x.astype(jnp.int32)
            y0 = y.astype(jnp.int32)
            x0c = jnp.clip(x0, 0, W - 1)
            x1c = jnp.clip(x0 + 1, 0, W - 1)
            y0c = jnp.clip(y0, 0, H - 1)
            y1c = jnp.clip(y0 + 1, 0, H - 1)
            x0f = x0c.astype(jnp.float32)
            x1f = x1c.astype(jnp.float32)
            y0f = y0c.astype(jnp.float32)
            y1f = y1c.astype(jnp.float32)
            wa_v[p, s16] = (x1f - x) * (y1f - y)
            wb_v[p, s16] = (x1f - x) * (y - y0f)
            wc_v[p, s16] = (x - x0f) * (y1f - y)
            wd_v[p, s16] = (x - x0f) * (y - y0f)
            row0 = batch_off + y0c * W
            row1 = batch_off + y1c * W
            ia = row0 + x0c
            ib = row1 + x0c
            ic = row0 + x1c
            idd = row1 + x1c
            id_v[p, s16] = idd
            pd_prev = plsc.load_gather(
                id_v.at[p], [jnp.maximum(iota + (g * L - 1), 0)])
            mxb = x1c != x0c
            myb = y1c != y0c
            mxyb = jnp.logical_and(mxb, myb)
            eab = ia != pd_prev
            if g == 0:
                eab = jnp.logical_or(eab, iota == 0)
            ea = eab.astype(jnp.int32)
            mx = mxb.astype(jnp.int32)
            my = myb.astype(jnp.int32)
            mxy = mx * my
            cnt = ea + mx + my + mxy
            incl = plsc.cumsum(cnt)
            s = (running + incl) - cnt
            wherea = (s + ea) - 1
            wherec = wherea + mx
            whereb = wherec + my
            whered = whereb + mxy
            plsc.store_scatter(cidx_v.at[p], [wherea], ia, mask=eab)
            plsc.store_scatter(cidx_v.at[p], [wherec], ic, mask=mxb)
            plsc.store_scatter(cidx_v.at[p], [whereb], ib, mask=myb)
            plsc.store_scatter(cidx_v.at[p], [whered], idd, mask=mxyb)
            pa_v[p, s16] = wherea
            pc_v[p, s16] = wherec
            pb_v[p, s16] = jnp.where(myb, whereb, wherea)
            pd_v[p, s16] = jnp.where(mxyb, whered,
                                     jnp.where(myb, whereb, wherec))
            running = running + jnp.max(incl)

        nch = (running + (CH - 1)) // CH

        def fire(j, c2):
            pltpu.async_copy(
                tab_hbm.at[cidx_v.at[p, pl.ds(j * CH, CH)]],
                crows_v.at[p, pl.ds(j * CH, CH)], gsems[p])
            return c2

        lax.fori_loop(0, nch, fire, 0)
        return nch

    def stage_drain(p, nch):
        def drain(j, c2):
            pltpu.make_async_copy(
                tab_hbm.at[cidx_v.at[p, pl.ds(j * CH, CH)]],
                crows_v.at[p, pl.ds(j * CH, CH)], gsems[p]).wait()
            return c2

        lax.fori_loop(0, nch, drain, 0)

    def stage_combine_write(blk, p):
        base = tile_base + blk * K

        @pl.when(blk >= 2)
        def _():
            pltpu.make_async_copy(
                out_v.at[p],
                out_hbm.at[pl.ds((base - 2 * K) * C, K * C)], wsem).wait()

        def px(i, c2):
            iv = jnp.full((L,), i, jnp.int32)
            wa = plsc.load_gather(wa_v.at[p], [iv])
            wb = plsc.load_gather(wb_v.at[p], [iv])
            wc = plsc.load_gather(wc_v.at[p], [iv])
            wd = plsc.load_gather(wd_v.at[p], [iv])
            pa = plsc.load_gather(pa_v.at[p], [iv])
            pb = plsc.load_gather(pb_v.at[p], [iv])
            pc = plsc.load_gather(pc_v.at[p], [iv])
            pd = plsc.load_gather(pd_v.at[p], [iv])
            ob = i * C
            for hh in range(2):
                hv = jnp.full((L,), hh, jnp.int32)
                for ch in range(6):
                    cv = iota + (ch * L)
                    va = plsc.load_gather(crows_v.at[p], [pa, hv, cv])
                    vb = plsc.load_gather(crows_v.at[p], [pb, hv, cv])
                    vc = plsc.load_gather(crows_v.at[p], [pc, hv, cv])
                    vd = plsc.load_gather(crows_v.at[p], [pd, hv, cv])
                    out_v[p, pl.ds(ob + hh * 96 + ch * L, L)] = (
                        (wa * va + wb * vb) + wc * vc) + wd * vd
            return c2

        lax.fori_loop(0, K, px, 0)
        pltpu.async_copy(out_v.at[p],
                         out_hbm.at[pl.ds(base * C, K * C)], wsem)

    nch0 = stage_compute_fire(0, 0)

    def pair(q, carry):
        nch_cur = carry
        blk0 = q * 2
        nch_nxt = stage_compute_fire(blk0 + 1, 1)
        stage_drain(0, nch_cur)
        stage_combine_write(blk0, 0)
        nch_cur2 = stage_compute_fire(blk0 + 2, 0)
        stage_drain(1, nch_nxt)
        stage_combine_write(blk0 + 1, 1)
        return nch_cur2

    nch_last = lax.fori_loop(0, NB // 2, pair, nch0)
    stage_drain(0, nch_last)
    last_base = tile_base + (NB - 2) * K
    pltpu.make_async_copy(
        out_v.at[0], out_hbm.at[pl.ds(last_base * C, K * C)], wsem).wait()
    pltpu.make_async_copy(
        out_v.at[1], out_hbm.at[pl.ds((last_base + K) * C, K * C)],
        wsem).wait()


@functools.partial(jax.jit, static_argnames=())
def _sc_sample(xd, yd, tab):
    mesh = plsc.VectorSubcoreMesh(core_axis_name="c", subcore_axis_name="s")
    f = pl.kernel(
        _body,
        out_type=jax.ShapeDtypeStruct((P * C,), jnp.float32),
        mesh=mesh,
        compiler_params=pltpu.CompilerParams(
            needs_layout_passes=False, use_tc_tiling_on_sc=False),
        scratch_types=[
            pltpu.VMEM((2, K), jnp.float32),
            pltpu.VMEM((2, K), jnp.float32),
            pltpu.VMEM((2, K), jnp.int32),
            pltpu.VMEM((2, K), jnp.float32),
            pltpu.VMEM((2, K), jnp.float32),
            pltpu.VMEM((2, K), jnp.float32),
            pltpu.VMEM((2, K), jnp.float32),
            pltpu.VMEM((2, K), jnp.int32),
            pltpu.VMEM((2, K), jnp.int32),
            pltpu.VMEM((2, K), jnp.int32),
            pltpu.VMEM((2, K), jnp.int32),
            pltpu.VMEM((2, CAP), jnp.int32),
            pltpu.VMEM((2, CAP, 2, 96), jnp.float32),
            pltpu.VMEM((2, K * C), jnp.float32),
            pltpu.SemaphoreType.DMA,
            pltpu.SemaphoreType.DMA,
            pltpu.SemaphoreType.DMA,
        ],
    )
    return f(xd, yd, tab)


def kernel(X, transformation):
    x_lin = jnp.linspace(-1.0, 1.0, OW)
    y_lin = jnp.linspace(-1.0, 1.0, OH)
    xc, yc = jnp.meshgrid(x_lin, y_lin)
    xf = xc.reshape(-1)
    yf = yc.reshape(-1)
    ones = jnp.ones_like(xf)
    grid = jnp.concatenate([xf, yf, ones], axis=0)
    grids = jnp.tile(grid, (B,)).reshape(B, 3, OH * OW)
    theta = transformation.reshape(B, 3, 3)
    sampled = jnp.matmul(theta, grids)
    x = sampled[:, 0, :].reshape(-1)
    y = sampled[:, 1, :].reshape(-1)
    z = sampled[:, 2, :].reshape(-1) + 1e-06
    xd = (x / z).astype(jnp.float32)
    yd = (y / z).astype(jnp.float32)
    pad = jnp.zeros((2 * K,), jnp.float32)
    xdp = jnp.concatenate([xd, pad])
    ydp = jnp.concatenate([yd, pad])
    zz = transformation.reshape(-1)[0] * jnp.float32(0.0)
    tab = X.reshape(P, 2, 96) + zz
    out = _sc_sample(xdp, ydp, tab)
    return out.reshape(B, OH, OW, C) + zz


# speedup vs baseline: 1.0663x; 1.0663x over previous
import functools

import jax
import jax.numpy as jnp
from jax import lax
from jax.experimental import pallas as pl
from jax.experimental.pallas import tpu as pltpu
from jax.experimental.pallas import tpu_sc as plsc

OH = 224
OW = 224
H = 224
W = 224
C = 192
B = 8
P = B * OH * OW
NW = 32
PT = P // NW
K = 64
NB = PT // K
L = 16
CAP = 4 * K
CH = 32

assert K % L == 0 and CAP % CH == 0 and PT % K == 0 and NB % 2 == 0


def _body(xd_hbm, yd_hbm, tab_hbm, out_hbm,
          xd_v, yd_v, id_v, wa_v, wb_v, wc_v, wd_v,
          pa_v, pb_v, pc_v, pd_v,
          cidx_v, crows_v, out_v, gsem0, gsem1, wsem):
    wid = lax.axis_index("s") * 2 + lax.axis_index("c")
    tile_base = wid * PT
    batch_off = (wid // 4) * (H * W)
    iota = lax.iota(jnp.int32, L)
    gsems = (gsem0, gsem1)

    for p in range(2):
        for hp in range(2):
            for q in range(CAP // L):
                cidx_v[p, hp, pl.ds(q * L, L)] = jnp.zeros((L,), jnp.int32)

    def stage_compute_fire(blk, p):
        base = tile_base + blk * K
        pltpu.sync_copy(xd_hbm.at[pl.ds(base, K)], xd_v.at[p])
        pltpu.sync_copy(yd_hbm.at[pl.ds(base, K)], yd_v.at[p])
        running = jnp.int32(0)
        for g in range(K // L):
            s16 = pl.ds(g * L, L)
            x = 0.5 * (xd_v[p, s16] + 1.0) * 224.0
            y = 0.5 * (yd_v[p, s16] + 1.0) * 224.0
            x0 = x.astype(jnp.int32)
            y0 = y.astype(jnp.int32)
            x0c = jnp.clip(x0, 0, W - 1)
            x1c = jnp.clip(x0 + 1, 0, W - 1)
            y0c = jnp.clip(y0, 0, H - 1)
            y1c = jnp.clip(y0 + 1, 0, H - 1)
            x0f = x0c.astype(jnp.float32)
            x1f = x1c.astype(jnp.float32)
            y0f = y0c.astype(jnp.float32)
            y1f = y1c.astype(jnp.float32)
            wa_v[p, s16] = (x1f - x) * (y1f - y)
            wb_v[p, s16] = (x1f - x) * (y - y0f)
            wc_v[p, s16] = (x - x0f) * (y1f - y)
            wd_v[p, s16] = (x - x0f) * (y - y0f)
            row0 = batch_off + y0c * W
            row1 = batch_off + y1c * W
            ia = row0 + x0c
            ib = row1 + x0c
            ic = row0 + x1c
            idd = row1 + x1c
            id_v[p, s16] = idd
            pd_prev = plsc.load_gather(
                id_v.at[p], [jnp.maximum(iota + (g * L - 1), 0)])
            mxb = x1c != x0c
            myb = y1c != y0c
            mxyb = jnp.logical_and(mxb, myb)
            eab = ia != pd_prev
            if g == 0:
                eab = jnp.logical_or(eab, iota == 0)
            ea = eab.astype(jnp.int32)
            mx = mxb.astype(jnp.int32)
            my = myb.astype(jnp.int32)
            mxy = mx * my
            cnt = ea + mx + my + mxy
            incl = plsc.cumsum(cnt)
            s = (running + incl) - cnt
            wherea = (s + ea) - 1
            wherec = wherea + mx
            whereb = wherec + my
            whered = whereb + mxy
            ia2 = ia * 2
            ib2 = ib * 2
            ic2 = ic * 2
            idd2 = idd * 2
            plsc.store_scatter(cidx_v.at[p, 0], [wherea], ia2, mask=eab)
            plsc.store_scatter(cidx_v.at[p, 0], [wherec], ic2, mask=mxb)
            plsc.store_scatter(cidx_v.at[p, 0], [whereb], ib2, mask=myb)
            plsc.store_scatter(cidx_v.at[p, 0], [whered], idd2, mask=mxyb)
            plsc.store_scatter(cidx_v.at[p, 1], [wherea], ia2 + 1, mask=eab)
            plsc.store_scatter(cidx_v.at[p, 1], [wherec], ic2 + 1, mask=mxb)
            plsc.store_scatter(cidx_v.at[p, 1], [whereb], ib2 + 1, mask=myb)
            plsc.store_scatter(cidx_v.at[p, 1], [whered], idd2 + 1, mask=mxyb)
            pa_v[p, s16] = wherea
            pc_v[p, s16] = wherec
            pb_v[p, s16] = jnp.where(myb, whereb, wherea)
            pd_v[p, s16] = jnp.where(mxyb, whered,
                                     jnp.where(myb, whereb, wherec))
            running = running + jnp.max(incl)

        nch = (running + (CH - 1)) // CH

        def fire(j, c2):
            for hp in range(2):
                pltpu.async_copy(
                    tab_hbm.at[cidx_v.at[p, hp, pl.ds(j * CH, CH)]],
                    crows_v.at[p, hp, pl.ds(j * CH, CH)], gsems[p])
            return c2

        lax.fori_loop(0, nch, fire, 0)
        return nch

    def stage_drain(p, nch):
        def drain(j, c2):
            for hp in range(2):
                pltpu.make_async_copy(
                    tab_hbm.at[cidx_v.at[p, hp, pl.ds(j * CH, CH)]],
                    crows_v.at[p, hp, pl.ds(j * CH, CH)], gsems[p]).wait()
            return c2

        lax.fori_loop(0, nch, drain, 0)

    def stage_combine_write(blk, p):
        base = tile_base + blk * K

        @pl.when(blk >= 2)
        def _():
            pltpu.make_async_copy(
                out_v.at[p],
                out_hbm.at[pl.ds((base - 2 * K) * 2, K * 2)], wsem).wait()

        def px(i, c2):
            iv = jnp.full((L,), i, jnp.int32)
            wa = plsc.load_gather(wa_v.at[p], [iv])
            wb = plsc.load_gather(wb_v.at[p], [iv])
            wc = plsc.load_gather(wc_v.at[p], [iv])
            wd = plsc.load_gather(wd_v.at[p], [iv])
            pa = plsc.load_gather(pa_v.at[p], [iv])
            pb = plsc.load_gather(pb_v.at[p], [iv])
            pc = plsc.load_gather(pc_v.at[p], [iv])
            pd = plsc.load_gather(pd_v.at[p], [iv])
            for hh in range(2):
                hv = jnp.full((L,), hh, jnp.int32)
                for ch in range(6):
                    cv = iota + (ch * L)
                    va = plsc.load_gather(crows_v.at[p], [hv, pa, cv])
                    vb = plsc.load_gather(crows_v.at[p], [hv, pb, cv])
                    vc = plsc.load_gather(crows_v.at[p], [hv, pc, cv])
                    vd = plsc.load_gather(crows_v.at[p], [hv, pd, cv])
                    out_v[p, i * 2 + hh, pl.ds(ch * L, L)] = (
                        (wa * va + wb * vb) + wc * vc) + wd * vd
            return c2

        lax.fori_loop(0, K, px, 0)
        pltpu.async_copy(out_v.at[p],
                         out_hbm.at[pl.ds(base * 2, K * 2)], wsem)

    nch0 = stage_compute_fire(0, 0)

    def pair(q, carry):
        nch_cur = carry
        blk0 = q * 2
        nch_nxt = stage_compute_fire(blk0 + 1, 1)
        stage_drain(0, nch_cur)
        stage_combine_write(blk0, 0)
        nch_cur2 = stage_compute_fire(blk0 + 2, 0)
        stage_drain(1, nch_nxt)
        stage_combine_write(blk0 + 1, 1)
        return nch_cur2

    nch_last = lax.fori_loop(0, NB // 2, pair, nch0)
    stage_drain(0, nch_last)
    last_base = tile_base + (NB - 2) * K
    pltpu.make_async_copy(
        out_v.at[0], out_hbm.at[pl.ds(last_base * 2, K * 2)], wsem).wait()
    pltpu.make_async_copy(
        out_v.at[1], out_hbm.at[pl.ds((last_base + K) * 2, K * 2)],
        wsem).wait()


@functools.partial(jax.jit, static_argnames=())
def _sc_sample(xd, yd, tab):
    mesh = plsc.VectorSubcoreMesh(core_axis_name="c", subcore_axis_name="s")
    f = pl.kernel(
        _body,
        out_type=jax.ShapeDtypeStruct((P * 2, 96), jnp.float32),
        mesh=mesh,
        compiler_params=pltpu.CompilerParams(
            needs_layout_passes=False, use_tc_tiling_on_sc=False),
        scratch_types=[
            pltpu.VMEM((2, K), jnp.float32),
            pltpu.VMEM((2, K), jnp.float32),
            pltpu.VMEM((2, K), jnp.int32),
            pltpu.VMEM((2, K), jnp.float32),
            pltpu.VMEM((2, K), jnp.float32),
            pltpu.VMEM((2, K), jnp.float32),
            pltpu.VMEM((2, K), jnp.float32),
            pltpu.VMEM((2, K), jnp.int32),
            pltpu.VMEM((2, K), jnp.int32),
            pltpu.VMEM((2, K), jnp.int32),
            pltpu.VMEM((2, K), jnp.int32),
            pltpu.VMEM((2, 2, CAP), jnp.int32),
            pltpu.VMEM((2, 2, CAP, 96), jnp.float32),
            pltpu.VMEM((2, K * 2, 96), jnp.float32),
            pltpu.SemaphoreType.DMA,
            pltpu.SemaphoreType.DMA,
            pltpu.SemaphoreType.DMA,
        ],
    )
    return f(xd, yd, tab)


def kernel(X, transformation):
    x_lin = jnp.linspace(-1.0, 1.0, OW)
    y_lin = jnp.linspace(-1.0, 1.0, OH)
    xc, yc = jnp.meshgrid(x_lin, y_lin)
    xf = xc.reshape(-1)
    yf = yc.reshape(-1)
    ones = jnp.ones_like(xf)
    grid = jnp.concatenate([xf, yf, ones], axis=0)
    grids = jnp.tile(grid, (B,)).reshape(B, 3, OH * OW)
    theta = transformation.reshape(B, 3, 3)
    sampled = jnp.matmul(theta, grids)
    x = sampled[:, 0, :].reshape(-1)
    y = sampled[:, 1, :].reshape(-1)
    z = sampled[:, 2, :].reshape(-1) + 1e-06
    xd = (x / z).astype(jnp.float32)
    yd = (y / z).astype(jnp.float32)
    pad = jnp.zeros((2 * K,), jnp.float32)
    xdp = jnp.concatenate([xd, pad])
    ydp = jnp.concatenate([yd, pad])
    tab = X.reshape(P * 2, 96).astype(jnp.float32)
    out = _sc_sample(xdp, ydp, tab)
    return out.reshape(B, OH, OW, C)


# speedup vs baseline: 1.0775x; 1.0105x over previous
import functools

import jax
import jax.numpy as jnp
from jax import lax
from jax.experimental import pallas as pl
from jax.experimental.pallas import tpu as pltpu
from jax.experimental.pallas import tpu_sc as plsc

OH = 224
OW = 224
H = 224
W = 224
C = 192
B = 8
P = B * OH * OW
NW = 32
PT = P // NW
K = 64
NB = PT // K
L = 16
CAP = 4 * K
CH = 32

assert K % L == 0 and CAP % CH == 0 and PT % K == 0 and NB % 2 == 0


def _body(xd_hbm, yd_hbm, tab_hbm, out_hbm,
          xd_v, yd_v, id_v, wa_v, wb_v, wc_v, wd_v,
          pa_v, pb_v, pc_v, pd_v,
          cidx_v, crows_v, out_v, gsem0, gsem1, wsem):
    wid = lax.axis_index("s") * 2 + lax.axis_index("c")
    tile_base = wid * PT
    batch_off = (wid // 4) * (H * W)
    iota = lax.iota(jnp.int32, L)
    gsems = (gsem0, gsem1)

    for p in range(2):
        for q in range(CAP // L):
            cidx_v[p, pl.ds(q * L, L)] = jnp.zeros((L,), jnp.int32)

    def stage_compute_fire(blk, p):
        base = tile_base + blk * K
        pltpu.sync_copy(xd_hbm.at[pl.ds(base, K)], xd_v.at[p])
        pltpu.sync_copy(yd_hbm.at[pl.ds(base, K)], yd_v.at[p])
        running = jnp.int32(0)
        for g in range(K // L):
            s16 = pl.ds(g * L, L)
            x = 0.5 * (xd_v[p, s16] + 1.0) * 224.0
            y = 0.5 * (yd_v[p, s16] + 1.0) * 224.0
            x0 = x.astype(jnp.int32)
            y0 = y.astype(jnp.int32)
            x0c = jnp.clip(x0, 0, W - 1)
            x1c = jnp.clip(x0 + 1, 0, W - 1)
            y0c = jnp.clip(y0, 0, H - 1)
            y1c = jnp.clip(y0 + 1, 0, H - 1)
            x0f = x0c.astype(jnp.float32)
            x1f = x1c.astype(jnp.float32)
            y0f = y0c.astype(jnp.float32)
            y1f = y1c.astype(jnp.float32)
            wa_v[p, s16] = (x1f - x) * (y1f - y)
            wb_v[p, s16] = (x1f - x) * (y - y0f)
            wc_v[p, s16] = (x - x0f) * (y1f - y)
            wd_v[p, s16] = (x - x0f) * (y - y0f)
            row0 = batch_off + y0c * W
            row1 = batch_off + y1c * W
            ia = row0 + x0c
            ib = row1 + x0c
            ic = row0 + x1c
            idd = row1 + x1c
            id_v[p, s16] = idd
            pd_prev = plsc.load_gather(
                id_v.at[p], [jnp.maximum(iota + (g * L - 1), 0)])
            mxb = x1c != x0c
            myb = y1c != y0c
            mxyb = jnp.logical_and(mxb, myb)
            eab = ia != pd_prev
            if g == 0:
                eab = jnp.logical_or(eab, iota == 0)
            ea = eab.astype(jnp.int32)
            mx = mxb.astype(jnp.int32)
            my = myb.astype(jnp.int32)
            mxy = mx * my
            cnt = ea + mx + my + mxy
            incl = plsc.cumsum(cnt)
            s = (running + incl) - cnt
            wherea = (s + ea) - 1
            wherec = wherea + mx
            whereb = wherec + my
            whered = whereb + mxy
            plsc.store_scatter(cidx_v.at[p], [wherea], ia, mask=eab)
            plsc.store_scatter(cidx_v.at[p], [wherec], ic, mask=mxb)
            plsc.store_scatter(cidx_v.at[p], [whereb], ib, mask=myb)
            plsc.store_scatter(cidx_v.at[p], [whered], idd, mask=mxyb)
            pa_v[p, s16] = wherea
            pc_v[p, s16] = wherec
            pb_v[p, s16] = jnp.where(myb, whereb, wherea)
            pd_v[p, s16] = jnp.where(mxyb, whered,
                                     jnp.where(myb, whereb, wherec))
            running = running + jnp.max(incl)

        nch = (running + (CH - 1)) // CH

        def fire(j, c2):
            pltpu.async_copy(
                tab_hbm.at[cidx_v.at[p, pl.ds(j * CH, CH)]],
                crows_v.at[p, pl.ds(j * CH, CH)], gsems[p])
            return c2

        lax.fori_loop(0, nch, fire, 0)
        return nch

    def stage_drain(p, nch):
        def drain(j, c2):
            pltpu.make_async_copy(
                tab_hbm.at[cidx_v.at[p, pl.ds(j * CH, CH)]],
                crows_v.at[p, pl.ds(j * CH, CH)], gsems[p]).wait()
            return c2

        lax.fori_loop(0, nch, drain, 0)

    def stage_combine_write(blk, p):
        base = tile_base + blk * K

        @pl.when(blk >= 2)
        def _():
            pltpu.make_async_copy(
                out_v.at[p],
                out_hbm.at[pl.ds((base - 2 * K) * C, K * C)], wsem).wait()

        def px(i, c2):
            iv = jnp.full((L,), i, jnp.int32)
            wa = plsc.load_gather(wa_v.at[p], [iv])
            wb = plsc.load_gather(wb_v.at[p], [iv])
            wc = plsc.load_gather(wc_v.at[p], [iv])
            wd = plsc.load_gather(wd_v.at[p], [iv])
            pa = plsc.load_gather(pa_v.at[p], [iv])
            pb = plsc.load_gather(pb_v.at[p], [iv])
            pc = plsc.load_gather(pc_v.at[p], [iv])
            pd = plsc.load_gather(pd_v.at[p], [iv])
            ob = i * C
            for hh in range(2):
                hv = jnp.full((L,), hh, jnp.int32)
                for ch in range(6):
                    cv = iota + (ch * L)
                    va = plsc.load_gather(crows_v.at[p], [pa, hv, cv])
                    vb = plsc.load_gather(crows_v.at[p], [pb, hv, cv])
                    vc = plsc.load_gather(crows_v.at[p], [pc, hv, cv])
                    vd = plsc.load_gather(crows_v.at[p], [pd, hv, cv])
                    out_v[p, pl.ds(ob + hh * 96 + ch * L, L)] = (
                        (wa * va + wb * vb) + wc * vc) + wd * vd
            return c2

        lax.fori_loop(0, K, px, 0)
        pltpu.async_copy(out_v.at[p],
                         out_hbm.at[pl.ds(base * C, K * C)], wsem)

    nch0 = stage_compute_fire(0, 0)

    def pair(q, carry):
        nch_cur = carry
        blk0 = q * 2
        nch_nxt = stage_compute_fire(blk0 + 1, 1)
        stage_drain(0, nch_cur)
        stage_combine_write(blk0, 0)
        nch_cur2 = stage_compute_fire(blk0 + 2, 0)
        stage_drain(1, nch_nxt)
        stage_combine_write(blk0 + 1, 1)
        return nch_cur2

    nch_last = lax.fori_loop(0, NB // 2, pair, nch0)
    stage_drain(0, nch_last)
    last_base = tile_base + (NB - 2) * K
    pltpu.make_async_copy(
        out_v.at[0], out_hbm.at[pl.ds(last_base * C, K * C)], wsem).wait()
    pltpu.make_async_copy(
        out_v.at[1], out_hbm.at[pl.ds((last_base + K) * C, K * C)],
        wsem).wait()


@functools.partial(jax.jit, static_argnames=())
def _sc_sample(xd, yd, tab):
    mesh = plsc.VectorSubcoreMesh(core_axis_name="c", subcore_axis_name="s")
    f = pl.kernel(
        _body,
        out_type=jax.ShapeDtypeStruct((P * C,), jnp.float32),
        mesh=mesh,
        compiler_params=pltpu.CompilerParams(
            needs_layout_passes=False, use_tc_tiling_on_sc=False),
        scratch_types=[
            pltpu.VMEM((2, K), jnp.float32),
            pltpu.VMEM((2, K), jnp.float32),
            pltpu.VMEM((2, K), jnp.int32),
            pltpu.VMEM((2, K), jnp.float32),
            pltpu.VMEM((2, K), jnp.float32),
            pltpu.VMEM((2, K), jnp.float32),
            pltpu.VMEM((2, K), jnp.float32),
            pltpu.VMEM((2, K), jnp.int32),
            pltpu.VMEM((2, K), jnp.int32),
            pltpu.VMEM((2, K), jnp.int32),
            pltpu.VMEM((2, K), jnp.int32),
            pltpu.VMEM((2, CAP), jnp.int32),
            pltpu.VMEM((2, CAP, 2, 96), jnp.float32),
            pltpu.VMEM((2, K * C), jnp.float32),
            pltpu.SemaphoreType.DMA,
            pltpu.SemaphoreType.DMA,
            pltpu.SemaphoreType.DMA,
        ],
    )
    return f(xd, yd, tab)


def kernel(X, transformation):
    x_lin = jnp.linspace(-1.0, 1.0, OW)
    y_lin = jnp.linspace(-1.0, 1.0, OH)
    xc, yc = jnp.meshgrid(x_lin, y_lin)
    xf = xc.reshape(-1)
    yf = yc.reshape(-1)
    ones = jnp.ones_like(xf)
    grid = jnp.concatenate([xf, yf, ones], axis=0)
    grids = jnp.tile(grid, (B,)).reshape(B, 3, OH * OW)
    theta = transformation.reshape(B, 3, 3)
    sampled = jnp.matmul(theta, grids)
    x = sampled[:, 0, :].reshape(-1)
    y = sampled[:, 1, :].reshape(-1)
    z = sampled[:, 2, :].reshape(-1) + 1e-06
    xd = (x / z).astype(jnp.float32)
    yd = (y / z).astype(jnp.float32)
    pad = jnp.zeros((2 * K,), jnp.float32)
    xdp = jnp.concatenate([xd, pad])
    ydp = jnp.concatenate([yd, pad])
    tab = X.reshape(P, 2, 96).astype(jnp.float32)
    out = _sc_sample(xdp, ydp, tab)
    return out.reshape(B, OH, OW, C)


# speedup vs baseline: 1.0788x; 1.0012x over previous
import functools

import jax
import jax.numpy as jnp
from jax import lax
from jax.experimental import pallas as pl
from jax.experimental.pallas import tpu as pltpu
from jax.experimental.pallas import tpu_sc as plsc

OH = 224
OW = 224
H = 224
W = 224
C = 192
B = 8
P = B * OH * OW
NW = 32
PT = P // NW
K = 64
NB = PT // K
L = 16
CAP = 4 * K
CH = 32

assert K % L == 0 and CAP % CH == 0 and PT % K == 0 and NB % 2 == 0


def _body(xd_hbm, yd_hbm, tab_hbm, out_hbm,
          xd_v, yd_v, id_v, wa_v, wb_v, wc_v, wd_v,
          pa_v, pb_v, pc_v, pd_v,
          cidx_v, crows_v, out_v, gsem0, gsem1, wsem0, wsem1):
    wid = lax.axis_index("s") * 2 + lax.axis_index("c")
    tile_base = wid * PT
    batch_off = (wid // 4) * (H * W)
    iota = lax.iota(jnp.int32, L)
    gsems = (gsem0, gsem1)
    wsems = (wsem0, wsem1)

    for p in range(2):
        for q in range(CAP // L):
            cidx_v[p, pl.ds(q * L, L)] = jnp.zeros((L,), jnp.int32)

    def stage_compute_fire(blk, p):
        base = tile_base + blk * K
        pltpu.sync_copy(xd_hbm.at[pl.ds(base, K)], xd_v.at[p])
        pltpu.sync_copy(yd_hbm.at[pl.ds(base, K)], yd_v.at[p])
        running = jnp.int32(0)
        for g in range(K // L):
            s16 = pl.ds(g * L, L)
            x = 0.5 * (xd_v[p, s16] + 1.0) * 224.0
            y = 0.5 * (yd_v[p, s16] + 1.0) * 224.0
            x0 = x.astype(jnp.int32)
            y0 = y.astype(jnp.int32)
            x0c = jnp.clip(x0, 0, W - 1)
            x1c = jnp.clip(x0 + 1, 0, W - 1)
            y0c = jnp.clip(y0, 0, H - 1)
            y1c = jnp.clip(y0 + 1, 0, H - 1)
            x0f = x0c.astype(jnp.float32)
            x1f = x1c.astype(jnp.float32)
            y0f = y0c.astype(jnp.float32)
            y1f = y1c.astype(jnp.float32)
            wa_v[p, s16] = (x1f - x) * (y1f - y)
            wb_v[p, s16] = (x1f - x) * (y - y0f)
            wc_v[p, s16] = (x - x0f) * (y1f - y)
            wd_v[p, s16] = (x - x0f) * (y - y0f)
            row0 = batch_off + y0c * W
            row1 = batch_off + y1c * W
            ia = row0 + x0c
            ib = row1 + x0c
            ic = row0 + x1c
            idd = row1 + x1c
            id_v[p, s16] = idd
            pd_prev = plsc.load_gather(
                id_v.at[p], [jnp.maximum(iota + (g * L - 1), 0)])
            mxb = x1c != x0c
            myb = y1c != y0c
            mxyb = jnp.logical_and(mxb, myb)
            eab = ia != pd_prev
            if g == 0:
                eab = jnp.logical_or(eab, iota == 0)
            ea = eab.astype(jnp.int32)
            mx = mxb.astype(jnp.int32)
            my = myb.astype(jnp.int32)
            mxy = mx * my
            cnt = ea + mx + my + mxy
            incl = plsc.cumsum(cnt)
            s = (running + incl) - cnt
            wherea = (s + ea) - 1
            wherec = wherea + mx
            whereb = wherec + my
            whered = whereb + mxy
            plsc.store_scatter(cidx_v.at[p], [wherea], ia, mask=eab)
            plsc.store_scatter(cidx_v.at[p], [wherec], ic, mask=mxb)
            plsc.store_scatter(cidx_v.at[p], [whereb], ib, mask=myb)
            plsc.store_scatter(cidx_v.at[p], [whered], idd, mask=mxyb)
            pa_v[p, s16] = wherea
            pc_v[p, s16] = wherec
            pb_v[p, s16] = jnp.where(myb, whereb, wherea)
            pd_v[p, s16] = jnp.where(mxyb, whered,
                                     jnp.where(myb, whereb, wherec))
            running = running + jnp.max(incl)

        nch = (running + (CH - 1)) // CH

        def fire(j, c2):
            pltpu.async_copy(
                tab_hbm.at[cidx_v.at[p, pl.ds(j * CH, CH)]],
                crows_v.at[p, pl.ds(j * CH, CH)], gsems[p])
            return c2

        lax.fori_loop(0, nch, fire, 0)
        return nch

    def stage_drain(p, nch):
        def drain(j, c2):
            pltpu.make_async_copy(
                tab_hbm.at[cidx_v.at[p, pl.ds(j * CH, CH)]],
                crows_v.at[p, pl.ds(j * CH, CH)], gsems[p]).wait()
            return c2

        lax.fori_loop(0, nch, drain, 0)

    def stage_combine_write(blk, p):
        base = tile_base + blk * K

        @pl.when(blk >= 2)
        def _():
            pltpu.make_async_copy(
                out_v.at[p],
                out_hbm.at[pl.ds((base - 2 * K) * C, K * C)], wsems[p]).wait()

        def px(i, c2):
            iv = jnp.full((L,), i, jnp.int32)
            wa = plsc.load_gather(wa_v.at[p], [iv])
            wb = plsc.load_gather(wb_v.at[p], [iv])
            wc = plsc.load_gather(wc_v.at[p], [iv])
            wd = plsc.load_gather(wd_v.at[p], [iv])
            pa = plsc.load_gather(pa_v.at[p], [iv])
            pb = plsc.load_gather(pb_v.at[p], [iv])
            pc = plsc.load_gather(pc_v.at[p], [iv])
            pd = plsc.load_gather(pd_v.at[p], [iv])
            ob = i * C
            for hh in range(2):
                hv = jnp.full((L,), hh, jnp.int32)
                for ch in range(6):
                    cv = iota + (ch * L)
                    va = plsc.load_gather(crows_v.at[p], [pa, hv, cv])
                    vb = plsc.load_gather(crows_v.at[p], [pb, hv, cv])
                    vc = plsc.load_gather(crows_v.at[p], [pc, hv, cv])
                    vd = plsc.load_gather(crows_v.at[p], [pd, hv, cv])
                    out_v[p, pl.ds(ob + hh * 96 + ch * L, L)] = (
                        (wa * va + wb * vb) + wc * vc) + wd * vd
            return c2

        lax.fori_loop(0, K, px, 0)
        pltpu.async_copy(out_v.at[p],
                         out_hbm.at[pl.ds(base * C, K * C)], wsems[p])

    nch0 = stage_compute_fire(0, 0)

    def pair(q, carry):
        nch_cur = carry
        blk0 = q * 2
        nch_nxt = stage_compute_fire(blk0 + 1, 1)
        stage_drain(0, nch_cur)
        stage_combine_write(blk0, 0)
        nch_cur2 = stage_compute_fire(blk0 + 2, 0)
        stage_drain(1, nch_nxt)
        stage_combine_write(blk0 + 1, 1)
        return nch_cur2

    nch_last = lax.fori_loop(0, NB // 2, pair, nch0)
    stage_drain(0, nch_last)
    last_base = tile_base + (NB - 2) * K
    pltpu.make_async_copy(
        out_v.at[0], out_hbm.at[pl.ds(last_base * C, K * C)], wsem0).wait()
    pltpu.make_async_copy(
        out_v.at[1], out_hbm.at[pl.ds((last_base + K) * C, K * C)],
        wsem1).wait()


@functools.partial(jax.jit, static_argnames=())
def _sc_sample(xd, yd, tab):
    mesh = plsc.VectorSubcoreMesh(core_axis_name="c", subcore_axis_name="s")
    f = pl.kernel(
        _body,
        out_type=jax.ShapeDtypeStruct((P * C,), jnp.float32),
        mesh=mesh,
        compiler_params=pltpu.CompilerParams(
            needs_layout_passes=False, use_tc_tiling_on_sc=False),
        scratch_types=[
            pltpu.VMEM((2, K), jnp.float32),
            pltpu.VMEM((2, K), jnp.float32),
            pltpu.VMEM((2, K), jnp.int32),
            pltpu.VMEM((2, K), jnp.float32),
            pltpu.VMEM((2, K), jnp.float32),
            pltpu.VMEM((2, K), jnp.float32),
            pltpu.VMEM((2, K), jnp.float32),
            pltpu.VMEM((2, K), jnp.int32),
            pltpu.VMEM((2, K), jnp.int32),
            pltpu.VMEM((2, K), jnp.int32),
            pltpu.VMEM((2, K), jnp.int32),
            pltpu.VMEM((2, CAP), jnp.int32),
            pltpu.VMEM((2, CAP, 2, 96), jnp.float32),
            pltpu.VMEM((2, K * C), jnp.float32),
            pltpu.SemaphoreType.DMA,
            pltpu.SemaphoreType.DMA,
            pltpu.SemaphoreType.DMA,
            pltpu.SemaphoreType.DMA,
        ],
    )
    return f(xd, yd, tab)


def kernel(X, transformation):
    x_lin = jnp.linspace(-1.0, 1.0, OW)
    y_lin = jnp.linspace(-1.0, 1.0, OH)
    xc, yc = jnp.meshgrid(x_lin, y_lin)
    xf = xc.reshape(-1)
    yf = yc.reshape(-1)
    ones = jnp.ones_like(xf)
    grid = jnp.concatenate([xf, yf, ones], axis=0)
    grids = jnp.tile(grid, (B,)).reshape(B, 3, OH * OW)
    theta = transformation.reshape(B, 3, 3)
    sampled = jnp.matmul(theta, grids)
    x = sampled[:, 0, :].reshape(-1)
    y = sampled[:, 1, :].reshape(-1)
    z = sampled[:, 2, :].reshape(-1) + 1e-06
    xd = (x / z).astype(jnp.float32)
    yd = (y / z).astype(jnp.float32)
    pad = jnp.zeros((2 * K,), jnp.float32)
    xdp = jnp.concatenate([xd, pad])
    ydp = jnp.concatenate([yd, pad])
    tab = X.reshape(P, 2, 96).astype(jnp.float32)
    out = _sc_sample(xdp, ydp, tab)
    return out.reshape(B, OH, OW, C)
